# baseline (speedup 1.0000x reference)
import jax
import jax.numpy as jnp
from jax import lax
from jax.experimental import pallas as pl
from jax.experimental.pallas import tpu as pltpu
from jax.experimental.pallas import tpu_sc as plsc

N_DRUG = 100000
N_SIDE = 100000
D = 128
TH = 24
B = 10000

NW = 32
BP = 10240
NPW = BP // NW
MISS = B + 16

_f32 = jnp.float32
_i32 = jnp.int32

_SC_MESH = plsc.VectorSubcoreMesh(core_axis_name="c", subcore_axis_name="s")


def _wid():
    return lax.axis_index("s") * 2 + lax.axis_index("c")


def _gather_nodes_body(nodes_hbm, adj_hbm, drug_hbm, neigh_out, drug_out,
                       idx_v, adj_v, row_v, sem_a, sem_d):
    base = _wid() * NPW
    pltpu.sync_copy(nodes_hbm.at[pl.ds(base, NPW)], idx_v)
    cp_a = pltpu.async_copy(adj_hbm.at[idx_v], adj_v, sem_a)
    cp_d = pltpu.async_copy(drug_hbm.at[idx_v], row_v, sem_d)
    cp_a.wait()
    pltpu.sync_copy(adj_v, neigh_out.at[pl.ds(base, NPW), :])
    cp_d.wait()
    pltpu.sync_copy(row_v, drug_out.at[pl.ds(base, NPW), :])


def _gather_nodes(nodes_pad, adj, drug_w):
    return pl.kernel(
        _gather_nodes_body,
        out_type=(
            jax.ShapeDtypeStruct((BP, TH), _i32),
            jax.ShapeDtypeStruct((BP, D), _f32),
        ),
        mesh=_SC_MESH,
        scratch_types=(
            pltpu.VMEM((NPW,), _i32),
            pltpu.VMEM((NPW, TH), _i32),
            pltpu.VMEM((NPW, D), _f32),
            pltpu.SemaphoreType.DMA,
            pltpu.SemaphoreType.DMA,
        ),
        compiler_params=pltpu.CompilerParams(use_tc_tiling_on_sc=False),
    )(nodes_pad, adj, drug_w)


IPW = BP * TH // NW
CH = 384
NCH = IPW // CH


def _gather_side_body(nidx_hbm, side_hbm, out_hbm,
                      idx0, idx1, buf0, buf1, sem0, sem1):
    base = _wid() * IPW
    idx = (idx0, idx1)
    buf = (buf0, buf1)
    sem = (sem0, sem1)

    def start(ci):
        k = ci % 2
        off = base + ci * CH
        pltpu.sync_copy(nidx_hbm.at[pl.ds(off, CH)], idx[k])
        return pltpu.async_copy(side_hbm.at[idx[k]], buf[k], sem[k])

    cp = start(0)
    for ci in range(NCH):
        nxt = start(ci + 1) if ci + 1 < NCH else None
        cp.wait()
        pltpu.sync_copy(buf[ci % 2], out_hbm.at[pl.ds(base + ci * CH, CH), :])
        cp = nxt


def _gather_side(neigh_flat, side_w):
    return pl.kernel(
        _gather_side_body,
        out_type=jax.ShapeDtypeStruct((BP * TH, D), _f32),
        mesh=_SC_MESH,
        scratch_types=(
            pltpu.VMEM((CH,), _i32),
            pltpu.VMEM((CH,), _i32),
            pltpu.VMEM((CH, D), _f32),
            pltpu.VMEM((CH, D), _f32),
            pltpu.SemaphoreType.DMA,
            pltpu.SemaphoreType.DMA,
        ),
    )(neigh_flat, side_w)


NB = 128
GRID = BP // NB


def _dense_body(drug, side, Wu, bu, Wi, bi, Wa1, ba1, wa2, ba2,
                nf_out, agg_out, wia_s, cb_s):
    i = pl.program_id(0)

    @pl.when(i == 0)
    def _():
        wia_s[...] = jnp.dot(Wi[...], Wa1[0:D, :], preferred_element_type=_f32)
        cb_s[...] = ba1[...] + jnp.dot(bi[...], Wa1[0:D, :],
                                       preferred_element_type=_f32)

    nf = jnp.dot(drug[...], Wu[...], preferred_element_type=_f32) + bu[...]
    nf_out[...] = nf
    cvec = jnp.dot(nf, Wa1[D:2 * D, :], preferred_element_type=_f32) + cb_s[...]
    G = jnp.dot(side[...], wia_s[...], preferred_element_type=_f32)
    h = jnp.maximum(G.reshape(NB, TH, D) + cvec[:, None, :], 0.0)
    s = jnp.sum(h * wa2[...][0][None, None, :], axis=-1) + ba2[0, 0]
    m = jnp.max(s, axis=-1, keepdims=True)
    e = jnp.exp(s - m)
    att = e / jnp.sum(e, axis=-1, keepdims=True)
    raw3 = side[...].reshape(NB, TH, D)
    ws = jnp.sum(att[:, :, None] * raw3, axis=1)
    agg = jnp.dot(ws, Wi[...], preferred_element_type=_f32) + bi[...]
    rows = i * NB + lax.broadcasted_iota(_i32, (NB, 1), 0)
    agg_out[...] = jnp.where(rows < B, agg, 0.0)


def _dense(drug_rows, side_rows, Wu, bu, Wi, bi, Wa1, ba1, wa2, ba2):
    full = lambda i: (0, 0)
    blk = lambda i: (i, 0)
    return pl.pallas_call(
        _dense_body,
        grid=(GRID,),
        in_specs=[
            pl.BlockSpec((NB, D), blk),
            pl.BlockSpec((NB * TH, D), blk),
            pl.BlockSpec((D, D), full),
            pl.BlockSpec((1, D), full),
            pl.BlockSpec((D, D), full),
            pl.BlockSpec((1, D), full),
            pl.BlockSpec((2 * D, D), full),
            pl.BlockSpec((1, D), full),
            pl.BlockSpec((1, D), full),
            pl.BlockSpec((1, 1), full),
        ],
        out_specs=[
            pl.BlockSpec((NB, D), blk),
            pl.BlockSpec((NB, D), blk),
        ],
        out_shape=[
            jax.ShapeDtypeStruct((BP, D), _f32),
            jax.ShapeDtypeStruct((BP, D), _f32),
        ],
        scratch_shapes=[
            pltpu.VMEM((D, D), _f32),
            pltpu.VMEM((1, D), _f32),
        ],
    )(drug_rows, side_rows, Wu, bu, Wi, bi, Wa1, ba1, wa2, ba2)


SW = 25
STR = N_DRUG // SW
CH3 = 800
NCH3 = STR // CH3


def _scatter_body(nodes_hbm, agg_hbm, out_hbm, nodes_v, pos_v, buf_v, sem):
    w = _wid()

    @pl.when(w < SW)
    def _():
        lo = w * STR
        pltpu.sync_copy(nodes_hbm, nodes_v)

        def init_body(k, carry):
            pos_v[pl.ds(k * 16, 16)] = jnp.full((16,), MISS, _i32)
            return carry

        lax.fori_loop(0, STR // 16, init_body, 0)

        def scan_body(j, carry):
            v = nodes_v[pl.ds(j * 16, 16)]
            local = v - lo
            msk = (local >= 0) & (local < STR)
            local = jnp.clip(local, 0, STR - 1)
            jid = j * 16 + lax.iota(_i32, 16)
            plsc.store_scatter(pos_v, [local], jid, mask=msk)
            return carry

        lax.fori_loop(0, B // 16, scan_body, 0)

        for ci in range(NCH3):
            cp = pltpu.async_copy(
                agg_hbm.at[pos_v.at[pl.ds(ci * CH3, CH3)]], buf_v, sem)
            cp.wait()
            pltpu.sync_copy(buf_v, out_hbm.at[pl.ds(lo + ci * CH3, CH3), :])


def _scatter(nodes, agg):
    return pl.kernel(
        _scatter_body,
        out_type=jax.ShapeDtypeStruct((N_DRUG, D), _f32),
        mesh=_SC_MESH,
        scratch_types=(
            pltpu.VMEM((B,), _i32),
            pltpu.VMEM((STR,), _i32),
            pltpu.VMEM((CH3, D), _f32),
            pltpu.SemaphoreType.DMA,
        ),
        compiler_params=pltpu.CompilerParams(needs_layout_passes=False),
    )(nodes, agg)


def kernel(nodes, adj_neighbors, drug_weight, side_weight,
           Wu, bu, Wi, bi, Wa1, ba1, Wa2, ba2):
    nodes = nodes.astype(_i32)
    nodes_pad = jnp.concatenate([nodes, jnp.zeros((BP - B,), _i32)])
    neigh2d, drug_rows = _gather_nodes(nodes_pad, adj_neighbors.astype(_i32),
                                       drug_weight)
    side_rows = _gather_side(neigh2d.reshape(-1), side_weight)
    nf_pad, agg = _dense(drug_rows, side_rows, Wu,
                         bu.reshape(1, D), Wi, bi.reshape(1, D),
                         Wa1, ba1.reshape(1, D),
                         Wa2.reshape(1, D), ba2.reshape(1, 1))
    embed = _scatter(nodes, agg)
    return nf_pad[:B], embed


# speedup vs baseline: 1.0113x; 1.0113x over previous
import jax
import jax.numpy as jnp
from jax import lax
from jax.experimental import pallas as pl
from jax.experimental.pallas import tpu as pltpu
from jax.experimental.pallas import tpu_sc as plsc

N_DRUG = 100000
N_SIDE = 100000
D = 128
TH = 24
B = 10000

NW = 32
BP = 10240
NPW = BP // NW
MISS = B + 16

_f32 = jnp.float32
_i32 = jnp.int32

def _sc_mesh():
    return plsc.VectorSubcoreMesh(core_axis_name="c", subcore_axis_name="s",
                                  num_cores=2, num_subcores=16)


def _wid():
    return lax.axis_index("s") * 2 + lax.axis_index("c")


def _gather_nodes_body(nodes_hbm, adj_hbm, drug_hbm, neigh_out, drug_out,
                       idx_v, adj_v, row_v, sem_a, sem_d):
    base = _wid() * NPW
    pltpu.sync_copy(nodes_hbm.at[pl.ds(base, NPW)], idx_v)
    cp_a = pltpu.async_copy(adj_hbm.at[idx_v], adj_v, sem_a)
    cp_d = pltpu.async_copy(drug_hbm.at[idx_v], row_v, sem_d)
    cp_a.wait()
    pltpu.sync_copy(adj_v, neigh_out.at[pl.ds(base, NPW), :])
    cp_d.wait()
    pltpu.sync_copy(row_v, drug_out.at[pl.ds(base, NPW), :])


def _gather_nodes(nodes_pad, adj, drug_w):
    return pl.kernel(
        _gather_nodes_body,
        out_type=(
            jax.ShapeDtypeStruct((BP, TH), _i32),
            jax.ShapeDtypeStruct((BP, D), _f32),
        ),
        mesh=_sc_mesh(),
        scratch_types=(
            pltpu.VMEM((NPW,), _i32),
            pltpu.VMEM((NPW, TH), _i32),
            pltpu.VMEM((NPW, D), _f32),
            pltpu.SemaphoreType.DMA,
            pltpu.SemaphoreType.DMA,
        ),
        compiler_params=pltpu.CompilerParams(use_tc_tiling_on_sc=False),
    )(nodes_pad, adj, drug_w)


IPW = BP * TH // NW
CH = 384
NCH = IPW // CH


def _gather_side_body(nidx_hbm, side_hbm, out_hbm,
                      idx0, idx1, buf0, buf1, sem0, sem1):
    base = _wid() * IPW
    idx = (idx0, idx1)
    buf = (buf0, buf1)
    sem = (sem0, sem1)

    def start(ci):
        k = ci % 2
        off = base + ci * CH
        pltpu.sync_copy(nidx_hbm.at[pl.ds(off, CH)], idx[k])
        return pltpu.async_copy(side_hbm.at[idx[k]], buf[k], sem[k])

    cp = start(0)
    for ci in range(NCH):
        nxt = start(ci + 1) if ci + 1 < NCH else None
        cp.wait()
        pltpu.sync_copy(buf[ci % 2], out_hbm.at[pl.ds(base + ci * CH, CH), :])
        cp = nxt


def _gather_side(neigh_flat, side_w):
    return pl.kernel(
        _gather_side_body,
        out_type=jax.ShapeDtypeStruct((BP * TH, D), _f32),
        mesh=_sc_mesh(),
        scratch_types=(
            pltpu.VMEM((CH,), _i32),
            pltpu.VMEM((CH,), _i32),
            pltpu.VMEM((CH, D), _f32),
            pltpu.VMEM((CH, D), _f32),
            pltpu.SemaphoreType.DMA,
            pltpu.SemaphoreType.DMA,
        ),
        compiler_params=pltpu.CompilerParams(use_tc_tiling_on_sc=False),
    )(neigh_flat, side_w)


NB = 128
GRID = BP // NB


def _dense_body(drug, side, Wu, bu, Wi, bi, Wa1, ba1, wa2,
                nf_out, agg_out, wia_s, cb_s, w2m_s):
    i = pl.program_id(0)

    @pl.when(i == 0)
    def _():
        wia_s[...] = jnp.dot(Wi[...], Wa1[0:D, :], preferred_element_type=_f32)
        cb_s[...] = ba1[...] + jnp.dot(bi[...], Wa1[0:D, :],
                                       preferred_element_type=_f32)
        w2m_s[...] = jnp.broadcast_to(wa2[...], (D, D))

    nf = jnp.dot(drug[...], Wu[...], preferred_element_type=_f32) + bu[...]
    nf_out[...] = nf
    cvec = jnp.dot(nf, Wa1[D:2 * D, :], preferred_element_type=_f32) + cb_s[...]
    G = jnp.dot(side[...], wia_s[...], preferred_element_type=_f32)
    h3 = jnp.maximum(G.reshape(NB, TH, D) + cvec[:, None, :], 0.0)
    sb3 = jnp.dot(h3.reshape(NB * TH, D), w2m_s[...],
                  preferred_element_type=_f32).reshape(NB, TH, D)
    m = jnp.max(sb3, axis=1, keepdims=True)
    e3 = jnp.exp(sb3 - m)
    den = jnp.sum(e3, axis=1)
    raw3 = side[...].reshape(NB, TH, D)
    ws = jnp.sum(e3 * raw3, axis=1) / den
    agg = jnp.dot(ws, Wi[...], preferred_element_type=_f32) + bi[...]
    rows = i * NB + lax.broadcasted_iota(_i32, (NB, 1), 0)
    agg_out[...] = jnp.where(rows < B, agg, 0.0)


def _dense(drug_rows, side_rows, Wu, bu, Wi, bi, Wa1, ba1, wa2):
    full = lambda i: (0, 0)
    blk = lambda i: (i, 0)
    return pl.pallas_call(
        _dense_body,
        grid=(GRID,),
        in_specs=[
            pl.BlockSpec((NB, D), blk),
            pl.BlockSpec((NB * TH, D), blk),
            pl.BlockSpec((D, D), full),
            pl.BlockSpec((1, D), full),
            pl.BlockSpec((D, D), full),
            pl.BlockSpec((1, D), full),
            pl.BlockSpec((2 * D, D), full),
            pl.BlockSpec((1, D), full),
            pl.BlockSpec((D, 1), full),
        ],
        out_specs=[
            pl.BlockSpec((NB, D), blk),
            pl.BlockSpec((NB, D), blk),
        ],
        out_shape=[
            jax.ShapeDtypeStruct((BP, D), _f32),
            jax.ShapeDtypeStruct((BP, D), _f32),
        ],
        scratch_shapes=[
            pltpu.VMEM((D, D), _f32),
            pltpu.VMEM((1, D), _f32),
            pltpu.VMEM((D, D), _f32),
        ],
    )(drug_rows, side_rows, Wu, bu, Wi, bi, Wa1, ba1, wa2)


SW = 25
STR = N_DRUG // SW
CH3 = 800
NCH3 = STR // CH3


def _scatter_body(nodes_hbm, agg_hbm, out_hbm, nodes_v, pos_v, buf_v, sem):
    w = _wid()

    @pl.when(w < SW)
    def _():
        lo = w * STR
        pltpu.sync_copy(nodes_hbm, nodes_v)

        def init_body(k, carry):
            pos_v[pl.ds(k * 16, 16)] = jnp.full((16,), MISS, _i32)
            return carry

        lax.fori_loop(0, STR // 16, init_body, 0)

        def scan_body(j, carry):
            v = nodes_v[pl.ds(j * 16, 16)]
            local = v - lo
            msk = (local >= 0) & (local < STR)
            local = jnp.clip(local, 0, STR - 1)
            jid = j * 16 + lax.iota(_i32, 16)
            plsc.store_scatter(pos_v, [local], jid, mask=msk)
            return carry

        lax.fori_loop(0, B // 16, scan_body, 0)

        for ci in range(NCH3):
            cp = pltpu.async_copy(
                agg_hbm.at[pos_v.at[pl.ds(ci * CH3, CH3)]], buf_v, sem)
            cp.wait()
            pltpu.sync_copy(buf_v, out_hbm.at[pl.ds(lo + ci * CH3, CH3), :])


def _scatter(nodes, agg):
    return pl.kernel(
        _scatter_body,
        out_type=jax.ShapeDtypeStruct((N_DRUG, D), _f32),
        mesh=_sc_mesh(),
        scratch_types=(
            pltpu.VMEM((B,), _i32),
            pltpu.VMEM((STR,), _i32),
            pltpu.VMEM((CH3, D), _f32),
            pltpu.SemaphoreType.DMA,
        ),
        compiler_params=pltpu.CompilerParams(needs_layout_passes=False,
                                             use_tc_tiling_on_sc=False),
    )(nodes, agg)


def kernel(nodes, adj_neighbors, drug_weight, side_weight,
           Wu, bu, Wi, bi, Wa1, ba1, Wa2, ba2):
    nodes = nodes.astype(_i32)
    nodes_pad = jnp.concatenate([nodes, jnp.zeros((BP - B,), _i32)])
    neigh2d, drug_rows = _gather_nodes(nodes_pad, adj_neighbors.astype(_i32),
                                       drug_weight)
    side_rows = _gather_side(neigh2d.reshape(-1), side_weight)
    nf_pad, agg = _dense(drug_rows, side_rows, Wu,
                         bu.reshape(1, D), Wi, bi.reshape(1, D),
                         Wa1, ba1.reshape(1, D), Wa2)
    embed = _scatter(nodes, agg)
    return nf_pad[:B], embed
